# Initial kernel scaffold; baseline (speedup 1.0000x reference)
import jax
import jax.numpy as jnp
from jax.experimental import pallas as pl


def kernel(features_batch, labels_batch):
    raise NotImplementedError("write your pallas kernel here")


# speedup vs baseline: 33.8493x; 33.8493x over previous
import jax
import jax.numpy as jnp
from jax.experimental import pallas as pl

_DD = 2.5
_GAMMA = 0.005
_K = 16
_HBLK = 48


def _stats_kernel(feat_ref, lab_ref, out_ref):
    h = pl.program_id(1)
    P = _HBLK * 384
    F = feat_ref[0]
    lab = lab_ref[0]

    classes = jax.lax.broadcasted_iota(jnp.int32, (_HBLK, 384, _K), 2)
    O = (lab[:, :, None] == classes).astype(jnp.float32)
    O2 = O.reshape(P, _K)

    f2 = jnp.sum(F * F, axis=0, keepdims=True)
    ones = jnp.ones((1, _HBLK, 384), dtype=jnp.float32)
    G = jnp.concatenate([F, f2, ones], axis=0)
    G2 = G.reshape(98, P)

    stats = jax.lax.dot_general(
        G2, O2, (((1,), (0,)), ((), ())),
        preferred_element_type=jnp.float32)

    @pl.when(h == 0)
    def _():
        out_ref[...] = jnp.zeros_like(out_ref)

    out_ref[0, 0:98, 0:_K] += stats


def _loss_kernel(stats_ref, cntcol_ref, out_ref):
    total = 0.0
    for b in range(4):
        st = stats_ref[b]
        sums = st[0:96, 0:_K]
        s2 = st[96:97, 0:_K]
        cnt = st[97:98, 0:_K]
        cnt_col = cntcol_ref[b]

        present = cnt > 0.0
        cnt_safe = jnp.maximum(cnt, 1.0)
        means = sums / cnt_safe
        m2 = jnp.sum(means * means, axis=0, keepdims=True)
        var_per = (s2 - cnt * m2) / cnt_safe
        var_loss = jnp.sum(jnp.where(present, var_per, 0.0))
        num_clusters = jnp.sum(present.astype(jnp.float32))

        diff = means[:, :, None] - means[:, None, :]
        d2 = jnp.sum(diff * diff, axis=0)
        ii = jax.lax.broadcasted_iota(jnp.int32, (_K, _K), 0)
        jj = jax.lax.broadcasted_iota(jnp.int32, (_K, _K), 1)
        pres_row = jnp.broadcast_to(cnt_col > 0.0, (_K, _K))
        pres_col = jnp.broadcast_to(present, (_K, _K))
        pair_mask = (ii < jj) & pres_row & pres_col
        dist = jnp.sqrt(jnp.where(pair_mask, d2, 1.0))
        denom = jnp.maximum(num_clusters - 1.0, 1.0)
        pen = jnp.where(pair_mask & (dist < 2.0 * _DD),
                        (2.0 * _DD - dist) ** 2 / denom, 0.0)
        dist_loss = jnp.where(num_clusters > 1.0, jnp.sum(pen), 0.0)

        mnorm = jnp.sqrt(jnp.where(present, m2, 1.0))
        reg_loss = jnp.sum(jnp.where(present, mnorm, 0.0))

        total = total + (var_loss + dist_loss + _GAMMA * reg_loss) / num_clusters

    out_ref[...] = jnp.broadcast_to(total / 5.0, (1, 1))


def kernel(features_batch, labels_batch):
    B, C, H, W = features_batch.shape
    stats = pl.pallas_call(
        _stats_kernel,
        grid=(B, H // _HBLK),
        in_specs=[
            pl.BlockSpec((1, C, _HBLK, W), lambda b, h: (b, 0, h, 0)),
            pl.BlockSpec((1, _HBLK, W), lambda b, h: (b, h, 0)),
        ],
        out_specs=pl.BlockSpec((1, 104, 128), lambda b, h: (b, 0, 0)),
        out_shape=jax.ShapeDtypeStruct((B, 104, 128), jnp.float32),
    )(features_batch, labels_batch)

    cntcol = stats[:, 97, 0:_K].reshape(B, _K, 1)
    loss = pl.pallas_call(
        _loss_kernel,
        out_shape=jax.ShapeDtypeStruct((1, 1), jnp.float32),
    )(stats, cntcol)
    return loss[0, 0]
